# baseline (speedup 1.0000x reference)
import functools

import jax
import jax.numpy as jnp
from jax import lax
from jax.experimental import pallas as pl
from jax.experimental.pallas import tpu as pltpu
from jax.experimental.pallas import tpu_sc as plsc

N_FIELDS = 26
VOCAB = 100000
DIM = 32
EPS = 1e-5
OUT_D = N_FIELDS * DIM

L = 16
NC = 2
NS = 16
NW = NC * NS

SEG_LO = (0, 33280, 66560)
SEG_LEN = (33280, 33280, VOCAB - 66560)
NSEG = 3
SEG_BUF = max(SEG_LEN)


def _make_sc_gather(B):
    assert DIM == NW
    NVEC = B // L
    UNROLL = 4
    mesh = plsc.VectorSubcoreMesh(core_axis_name="c", subcore_axis_name="s")

    @functools.partial(
        pl.kernel,
        mesh=mesh,
        compiler_params=pltpu.CompilerParams(
            needs_layout_passes=False, use_tc_tiling_on_sc=True),
        out_type=jax.ShapeDtypeStruct((OUT_D, B), jnp.float32),
        scratch_types=[
            pltpu.VMEM((SEG_BUF,), jnp.float32),
            pltpu.VMEM((SEG_BUF,), jnp.float32),
            pltpu.VMEM((B,), jnp.int32),
            pltpu.VMEM((B,), jnp.int32),
            pltpu.VMEM((B,), jnp.float32),
            pltpu.SemaphoreType.DMA,
            pltpu.SemaphoreType.DMA,
            pltpu.SemaphoreType.DMA,
        ],
    )
    def sc_gather(tabT, catT, out, seg_0, seg_1, idx_a, idx_b, res_v,
                  seg_sem, idx_sem, out_sem):
        d = lax.axis_index("s") * NC + lax.axis_index("c")
        segs = (seg_0, seg_1)

        def seg_copy(f, s, slot):
            return pltpu.make_async_copy(
                tabT.at[f, d, pl.ds(SEG_LO[s], SEG_LEN[s])],
                segs[slot].at[pl.ds(0, SEG_LEN[s])],
                seg_sem)

        def sweep(s, slot, idx_r):
            seg = segs[slot]
            lo = SEG_LO[s]
            hi = lo + SEG_LEN[s]

            def body(i, carry):
                for u in range(UNROLL):
                    j = i * UNROLL + u
                    sl = pl.ds(j * L, L)
                    iv = idx_r[sl]
                    if s == 0:
                        m = iv < hi
                        lv = jnp.minimum(iv, SEG_LEN[s] - 1)
                    elif s == NSEG - 1:
                        m = iv >= lo
                        lv = jnp.maximum(iv - lo, 0)
                    else:
                        m = (iv >= lo) & (iv < hi)
                        lv = jnp.minimum(
                            jnp.maximum(iv - lo, 0), SEG_LEN[s] - 1)
                    g = plsc.load_gather(seg, [lv])
                    res_v[sl] = jnp.where(m, g, res_v[sl])
                return carry

            lax.fori_loop(0, NVEC // UNROLL, body, 0)

        def field(f, idx_r, idx_next, spat, nslot0):
            @pl.when(f > 0)
            def _():
                pltpu.make_async_copy(res_v, out.at[0], out_sem).wait()

            @pl.when(f < N_FIELDS - 1)
            def _():
                pltpu.async_copy(catT.at[f + 1], idx_next, idx_sem)

            seg_copy(f, 0, spat[0]).wait()
            seg_copy(f, 1, spat[1]).start()
            pltpu.make_async_copy(catT.at[f], idx_r, idx_sem).wait()
            sweep(0, spat[0], idx_r)
            seg_copy(f, 2, spat[2]).start()
            seg_copy(f, 1, spat[1]).wait()
            sweep(1, spat[1], idx_r)

            @pl.when(f < N_FIELDS - 1)
            def _():
                seg_copy(f + 1, 0, nslot0).start()

            seg_copy(f, 2, spat[2]).wait()
            sweep(2, spat[2], idx_r)
            pltpu.async_copy(res_v, out.at[f * DIM + d], out_sem)

        pltpu.async_copy(catT.at[0], idx_a, idx_sem)
        seg_copy(0, 0, 0).start()

        def gbody(g, carry):
            field(2 * g, idx_a, idx_b, (0, 1, 0), 1)
            field(2 * g + 1, idx_b, idx_a, (1, 0, 1), 0)
            return carry

        lax.fori_loop(0, N_FIELDS // 2, gbody, 0)
        pltpu.make_async_copy(res_v, out.at[0], out_sem).wait()

    return sc_gather


def _tc_layernorm(gath, gamma, beta):
    D, B = gath.shape
    BL = 512

    def ln_body(x_ref, g_ref, b_ref, o_ref):
        x = x_ref[...]
        mean = jnp.mean(x, axis=0, keepdims=True)
        xc = x - mean
        var = jnp.mean(xc * xc, axis=0, keepdims=True)
        r = lax.rsqrt(var + EPS)
        o_ref[...] = xc * r * g_ref[...] + b_ref[...]

    return pl.pallas_call(
        ln_body,
        grid=(B // BL,),
        in_specs=[
            pl.BlockSpec((D, BL), lambda i: (0, i)),
            pl.BlockSpec((D, 1), lambda i: (0, 0)),
            pl.BlockSpec((D, 1), lambda i: (0, 0)),
        ],
        out_specs=pl.BlockSpec((D, BL), lambda i: (0, i)),
        out_shape=jax.ShapeDtypeStruct((D, B), jnp.float32),
    )(gath, gamma.reshape(D, 1), beta.reshape(D, 1))


def kernel(cat, tables, gamma, beta):
    B = cat.shape[0]
    catT = cat.T
    tabT = tables.transpose(0, 2, 1)
    gath = _make_sc_gather(B)(tabT, catT)
    outT = _tc_layernorm(gath, gamma, beta)
    return outT.T


# speedup vs baseline: 1.3154x; 1.3154x over previous
import functools

import jax
import jax.numpy as jnp
from jax import lax
from jax.experimental import pallas as pl
from jax.experimental.pallas import tpu as pltpu
from jax.experimental.pallas import tpu_sc as plsc

N_FIELDS = 26
VOCAB = 100000
DIM = 32
EPS = 1e-5
OUT_D = N_FIELDS * DIM

L = 16
NC = 2
NS = 16
NW = NC * NS

QI = 4


def _make_sc_gather(B):
    assert DIM == NW
    QB = B // QI
    QV = QB // L
    UNROLL = 4
    mesh = plsc.VectorSubcoreMesh(core_axis_name="c", subcore_axis_name="s")

    @functools.partial(
        pl.kernel,
        mesh=mesh,
        compiler_params=pltpu.CompilerParams(
            needs_layout_passes=False, use_tc_tiling_on_sc=True),
        out_type=jax.ShapeDtypeStruct((OUT_D, B), jnp.float32),
        scratch_types=[
            pltpu.VMEM((VOCAB,), jnp.float32),
            pltpu.VMEM((QB,), jnp.int32),
            pltpu.VMEM((QB,), jnp.int32),
            pltpu.VMEM((B,), jnp.float32),
            pltpu.SemaphoreType.DMA,
            pltpu.SemaphoreType.DMA,
            pltpu.SemaphoreType.DMA,
        ],
    )
    def sc_gather(tabT, catT, out, plane_v, idx_0, idx_1, res_v,
                  plane_sem, idx_sem, out_sem):
        d = lax.axis_index("s") * NC + lax.axis_index("c")
        idxs = (idx_0, idx_1)

        def plane_copy(f):
            return pltpu.make_async_copy(tabT.at[f, d], plane_v, plane_sem)

        def idx_copy(f, q, slot):
            return pltpu.make_async_copy(
                catT.at[f, pl.ds(q * QB, QB)], idxs[slot], idx_sem)

        def fbody(f, carry):
            @pl.when(f > 0)
            def _():
                pltpu.make_async_copy(res_v, out.at[0], out_sem).wait()

            plane_copy(f).wait()
            for q in range(QI):
                slot = q % 2
                idx_copy(f, q, slot).wait()
                if q < QI - 1:
                    idx_copy(f, q + 1, 1 - slot).start()
                else:
                    @pl.when(f < N_FIELDS - 1)
                    def _():
                        idx_copy(f + 1, 0, 1 - slot).start()

                def sweep(i, c2, q=q, slot=slot):
                    for u in range(UNROLL):
                        j = i * UNROLL + u
                        iv = idxs[slot][pl.ds(j * L, L)]
                        g = plsc.load_gather(plane_v, [iv])
                        res_v[pl.ds(q * QB + j * L, L)] = g
                    return c2

                lax.fori_loop(0, QV // UNROLL, sweep, 0)

            pltpu.async_copy(res_v, out.at[f * DIM + d], out_sem)

            @pl.when(f < N_FIELDS - 1)
            def _():
                plane_copy(f + 1).start()
            return carry

        pltpu.async_copy(catT.at[0, pl.ds(0, QB)], idx_0, idx_sem)
        plane_copy(0).start()
        lax.fori_loop(0, N_FIELDS, fbody, 0)
        pltpu.make_async_copy(res_v, out.at[0], out_sem).wait()

    return sc_gather


def _tc_layernorm(gath, gamma, beta):
    D, B = gath.shape
    BL = 512

    def ln_body(x_ref, g_ref, b_ref, o_ref):
        x = x_ref[...]
        mean = jnp.mean(x, axis=0, keepdims=True)
        xc = x - mean
        var = jnp.mean(xc * xc, axis=0, keepdims=True)
        r = lax.rsqrt(var + EPS)
        o_ref[...] = xc * r * g_ref[...] + b_ref[...]

    return pl.pallas_call(
        ln_body,
        grid=(B // BL,),
        in_specs=[
            pl.BlockSpec((D, BL), lambda i: (0, i)),
            pl.BlockSpec((D, 1), lambda i: (0, 0)),
            pl.BlockSpec((D, 1), lambda i: (0, 0)),
        ],
        out_specs=pl.BlockSpec((D, BL), lambda i: (0, i)),
        out_shape=jax.ShapeDtypeStruct((D, B), jnp.float32),
    )(gath, gamma.reshape(D, 1), beta.reshape(D, 1))


def kernel(cat, tables, gamma, beta):
    B = cat.shape[0]
    catT = cat.T
    tabT = tables.transpose(0, 2, 1)
    gath = _make_sc_gather(B)(tabT, catT)
    outT = _tc_layernorm(gath, gamma, beta)
    return outT.T


# speedup vs baseline: 1.8691x; 1.4210x over previous
import functools

import jax
import jax.numpy as jnp
from jax import lax
from jax.experimental import pallas as pl
from jax.experimental.pallas import tpu as pltpu
from jax.experimental.pallas import tpu_sc as plsc

N_FIELDS = 26
VOCAB = 100000
DIM = 32
EPS = 1e-5
OUT_D = N_FIELDS * DIM

L = 16
NC = 2
NS = 16
NW = NC * NS

QI = 4


def _make_sc_gather(B):
    assert DIM == NW
    QB = B // QI
    QV = QB // L
    UNROLL = 4
    mesh = plsc.VectorSubcoreMesh(core_axis_name="c", subcore_axis_name="s")

    @functools.partial(
        pl.kernel,
        mesh=mesh,
        compiler_params=pltpu.CompilerParams(
            needs_layout_passes=False, use_tc_tiling_on_sc=True),
        out_type=jax.ShapeDtypeStruct((OUT_D, B), jnp.float32),
        scratch_types=[
            pltpu.VMEM((VOCAB,), jnp.float32),
            pltpu.VMEM((QB,), jnp.int32),
            pltpu.VMEM((QB,), jnp.int32),
            pltpu.VMEM((B,), jnp.float32),
            pltpu.SemaphoreType.DMA,
            pltpu.SemaphoreType.DMA,
            pltpu.SemaphoreType.DMA,
        ],
    )
    def sc_gather(tabT, catT, out, plane_v, idx_0, idx_1, res_v,
                  plane_sem, idx_sem, out_sem):
        d = lax.axis_index("s") * NC + lax.axis_index("c")
        idxs = (idx_0, idx_1)

        def plane_copy(f):
            return pltpu.make_async_copy(tabT.at[f, d], plane_v, plane_sem)

        def idx_copy(f, q, slot):
            return pltpu.make_async_copy(
                catT.at[f, pl.ds(q * QB, QB)], idxs[slot], idx_sem)

        def fbody(f, carry):
            @pl.when(f > 0)
            def _():
                pltpu.make_async_copy(res_v, out.at[0], out_sem).wait()

            plane_copy(f).wait()
            for q in range(QI):
                slot = q % 2
                idx_copy(f, q, slot).wait()
                if q < QI - 1:
                    idx_copy(f, q + 1, 1 - slot).start()
                else:
                    @pl.when(f < N_FIELDS - 1)
                    def _():
                        idx_copy(f + 1, 0, 1 - slot).start()

                @plsc.parallel_loop(0, QV, unroll=UNROLL)
                def _(j, q=q, slot=slot):
                    iv = idxs[slot][pl.ds(j * L, L)]
                    g = plsc.load_gather(plane_v, [iv])
                    res_v[pl.ds(q * QB + j * L, L)] = g

            pltpu.async_copy(res_v, out.at[f * DIM + d], out_sem)

            @pl.when(f < N_FIELDS - 1)
            def _():
                plane_copy(f + 1).start()
            return carry

        pltpu.async_copy(catT.at[0, pl.ds(0, QB)], idx_0, idx_sem)
        plane_copy(0).start()
        lax.fori_loop(0, N_FIELDS, fbody, 0)
        pltpu.make_async_copy(res_v, out.at[0], out_sem).wait()

    return sc_gather


def _tc_layernorm(gath, gamma, beta):
    D, B = gath.shape
    BL = 512

    def ln_body(x_ref, g_ref, b_ref, o_ref):
        x = x_ref[...]
        mean = jnp.mean(x, axis=0, keepdims=True)
        xc = x - mean
        var = jnp.mean(xc * xc, axis=0, keepdims=True)
        r = lax.rsqrt(var + EPS)
        o_ref[...] = xc * r * g_ref[...] + b_ref[...]

    return pl.pallas_call(
        ln_body,
        grid=(B // BL,),
        in_specs=[
            pl.BlockSpec((D, BL), lambda i: (0, i)),
            pl.BlockSpec((D, 1), lambda i: (0, 0)),
            pl.BlockSpec((D, 1), lambda i: (0, 0)),
        ],
        out_specs=pl.BlockSpec((D, BL), lambda i: (0, i)),
        out_shape=jax.ShapeDtypeStruct((D, B), jnp.float32),
    )(gath, gamma.reshape(D, 1), beta.reshape(D, 1))


def kernel(cat, tables, gamma, beta):
    B = cat.shape[0]
    catT = cat.T
    tabT = tables.transpose(0, 2, 1)
    gath = _make_sc_gather(B)(tabT, catT)
    outT = _tc_layernorm(gath, gamma, beta)
    return outT.T


# speedup vs baseline: 1.9214x; 1.0280x over previous
import functools

import jax
import jax.numpy as jnp
from jax import lax
from jax.experimental import pallas as pl
from jax.experimental.pallas import tpu as pltpu
from jax.experimental.pallas import tpu_sc as plsc

N_FIELDS = 26
VOCAB = 100000
DIM = 32
EPS = 1e-5
OUT_D = N_FIELDS * DIM

L = 16
NC = 2
NS = 16
NW = NC * NS

QI = 4


def _make_sc_gather(B):
    assert DIM == NW
    QB = B // QI
    QV = QB // L
    UNROLL = 4
    mesh = plsc.VectorSubcoreMesh(core_axis_name="c", subcore_axis_name="s")

    @functools.partial(
        pl.kernel,
        mesh=mesh,
        compiler_params=pltpu.CompilerParams(
            needs_layout_passes=False, use_tc_tiling_on_sc=True),
        out_type=jax.ShapeDtypeStruct((OUT_D, B), jnp.float32),
        scratch_types=[
            pltpu.VMEM((VOCAB,), jnp.float32),
            pltpu.VMEM((QB,), jnp.int32),
            pltpu.VMEM((QB,), jnp.int32),
            pltpu.VMEM((B,), jnp.float32),
            pltpu.SemaphoreType.DMA,
            pltpu.SemaphoreType.DMA,
            pltpu.SemaphoreType.DMA,
        ],
    )
    def sc_gather(tabT, catT, out, plane_v, idx_0, idx_1, res_v,
                  plane_sem, idx_sem, out_sem):
        d = lax.axis_index("s") * NC + lax.axis_index("c")
        idxs = (idx_0, idx_1)

        def plane_copy(f):
            return pltpu.make_async_copy(tabT.at[f, d], plane_v, plane_sem)

        def idx_copy(f, q, slot):
            return pltpu.make_async_copy(
                catT.at[f, pl.ds(q * QB, QB)], idxs[slot], idx_sem)

        def fbody(f, carry):
            @pl.when(f > 0)
            def _():
                pltpu.make_async_copy(res_v, out.at[0], out_sem).wait()

            plane_copy(f).wait()
            for q in range(QI):
                slot = q % 2
                idx_copy(f, q, slot).wait()
                if q < QI - 1:
                    idx_copy(f, q + 1, 1 - slot).start()
                else:
                    @pl.when(f < N_FIELDS - 1)
                    def _():
                        idx_copy(f + 1, 0, 1 - slot).start()

                @plsc.parallel_loop(0, QV, unroll=UNROLL)
                def _(j, q=q, slot=slot):
                    iv = idxs[slot][pl.ds(j * L, L)]
                    g = plsc.load_gather(plane_v, [iv])
                    res_v[pl.ds(q * QB + j * L, L)] = g

            pltpu.async_copy(res_v, out.at[f * DIM + d], out_sem)

            @pl.when(f < N_FIELDS - 1)
            def _():
                plane_copy(f + 1).start()
            return carry

        pltpu.async_copy(catT.at[0, pl.ds(0, QB)], idx_0, idx_sem)
        plane_copy(0).start()
        lax.fori_loop(0, N_FIELDS, fbody, 0)
        pltpu.make_async_copy(res_v, out.at[0], out_sem).wait()

    return sc_gather


def _tc_layernorm(gath, gamma, beta):
    D, B = gath.shape
    BL = 1024

    def ln_body(x_ref, g_ref, b_ref, o_ref):
        x = x_ref[...]
        mean = jnp.mean(x, axis=0, keepdims=True)
        xc = x - mean
        var = jnp.mean(xc * xc, axis=0, keepdims=True)
        r = lax.rsqrt(var + EPS)
        o_ref[...] = xc * r * g_ref[...] + b_ref[...]

    return pl.pallas_call(
        ln_body,
        grid=(B // BL,),
        in_specs=[
            pl.BlockSpec((D, BL), lambda i: (0, i)),
            pl.BlockSpec((D, 1), lambda i: (0, 0)),
            pl.BlockSpec((D, 1), lambda i: (0, 0)),
        ],
        out_specs=pl.BlockSpec((D, BL), lambda i: (0, i)),
        out_shape=jax.ShapeDtypeStruct((D, B), jnp.float32),
    )(gath, gamma.reshape(D, 1), beta.reshape(D, 1))


def kernel(cat, tables, gamma, beta):
    B = cat.shape[0]
    catT = cat.T
    tabT = tables.transpose(0, 2, 1)
    gath = _make_sc_gather(B)(tabT, catT)
    outT = _tc_layernorm(gath, gamma, beta)
    return outT.T


# speedup vs baseline: 1.9327x; 1.0059x over previous
import functools

import jax
import jax.numpy as jnp
from jax import lax
from jax.experimental import pallas as pl
from jax.experimental.pallas import tpu as pltpu
from jax.experimental.pallas import tpu_sc as plsc

N_FIELDS = 26
VOCAB = 100000
DIM = 32
EPS = 1e-5
OUT_D = N_FIELDS * DIM

L = 16
NC = 2
NS = 16
NW = NC * NS

QI = 4


def _make_sc_gather(B):
    assert DIM == NW
    QB = B // QI
    QV = QB // L
    UNROLL = 4
    mesh = plsc.VectorSubcoreMesh(core_axis_name="c", subcore_axis_name="s")

    @functools.partial(
        pl.kernel,
        mesh=mesh,
        compiler_params=pltpu.CompilerParams(
            needs_layout_passes=False, use_tc_tiling_on_sc=True),
        out_type=jax.ShapeDtypeStruct((OUT_D, B), jnp.float32),
        scratch_types=[
            pltpu.VMEM((VOCAB,), jnp.float32),
            pltpu.VMEM((QB,), jnp.int32),
            pltpu.VMEM((QB,), jnp.int32),
            pltpu.VMEM((B,), jnp.float32),
            pltpu.SemaphoreType.DMA,
            pltpu.SemaphoreType.DMA,
            pltpu.SemaphoreType.DMA,
        ],
    )
    def sc_gather(tabT, catT, out, plane_v, idx_0, idx_1, res_v,
                  plane_sem, idx_sem, out_sem):
        d = lax.axis_index("s") * NC + lax.axis_index("c")
        idxs = (idx_0, idx_1)

        def plane_copy(f):
            return pltpu.make_async_copy(tabT.at[f, d], plane_v, plane_sem)

        def idx_copy(f, q, slot):
            return pltpu.make_async_copy(
                catT.at[f, pl.ds(q * QB, QB)], idxs[slot], idx_sem)

        def fbody(f, carry):
            @pl.when(f > 0)
            def _():
                pltpu.make_async_copy(res_v, out.at[0], out_sem).wait()

            plane_copy(f).wait()
            for q in range(QI):
                slot = q % 2
                idx_copy(f, q, slot).wait()
                if q < QI - 1:
                    idx_copy(f, q + 1, 1 - slot).start()
                else:
                    @pl.when(f < N_FIELDS - 1)
                    def _():
                        idx_copy(f + 1, 0, 1 - slot).start()

                @plsc.parallel_loop(0, QV, unroll=UNROLL)
                def _(j, q=q, slot=slot):
                    iv = idxs[slot][pl.ds(j * L, L)]
                    g = plsc.load_gather(plane_v, [iv])
                    res_v[pl.ds(q * QB + j * L, L)] = g

            pltpu.async_copy(res_v, out.at[f * DIM + d], out_sem)

            @pl.when(f < N_FIELDS - 1)
            def _():
                plane_copy(f + 1).start()
            return carry

        pltpu.async_copy(catT.at[0, pl.ds(0, QB)], idx_0, idx_sem)
        plane_copy(0).start()
        lax.fori_loop(0, N_FIELDS, fbody, 0)
        pltpu.make_async_copy(res_v, out.at[0], out_sem).wait()

    return sc_gather


def _tc_layernorm(gath, gamma, beta):
    D, B = gath.shape
    BL = 2048

    def ln_body(x_ref, g_ref, b_ref, o_ref):
        x = x_ref[...]
        mean = jnp.mean(x, axis=0, keepdims=True)
        xc = x - mean
        var = jnp.mean(xc * xc, axis=0, keepdims=True)
        r = lax.rsqrt(var + EPS)
        o_ref[...] = xc * r * g_ref[...] + b_ref[...]

    return pl.pallas_call(
        ln_body,
        grid=(B // BL,),
        in_specs=[
            pl.BlockSpec((D, BL), lambda i: (0, i)),
            pl.BlockSpec((D, 1), lambda i: (0, 0)),
            pl.BlockSpec((D, 1), lambda i: (0, 0)),
        ],
        out_specs=pl.BlockSpec((D, BL), lambda i: (0, i)),
        out_shape=jax.ShapeDtypeStruct((D, B), jnp.float32),
    )(gath, gamma.reshape(D, 1), beta.reshape(D, 1))


def kernel(cat, tables, gamma, beta):
    B = cat.shape[0]
    catT = cat.T
    tabT = tables.transpose(0, 2, 1)
    gath = _make_sc_gather(B)(tabT, catT)
    outT = _tc_layernorm(gath, gamma, beta)
    return outT.T
